# baseline (speedup 1.0000x reference)
import jax
import jax.numpy as jnp
from jax.experimental import pallas as pl
from jax.experimental.pallas import tpu as pltpu

_D = 1024
_E = 8
_H = 2048
_T = 1024
_B = 128
_NB = 23
_P = _NB * _B

_f32 = jnp.float32
_bf16 = jnp.bfloat16


def _iota(shape, dim):
    return jax.lax.broadcasted_iota(jnp.int32, shape, dim).astype(_f32)


def _router_body(xbf_ref, wr_ref, oh_ref, oht_ref, gate_ref, meta_ref):
    logits = jnp.dot(xbf_ref[...], wr_ref[...].astype(_bf16),
                     preferred_element_type=_f32)
    m = jnp.max(logits, axis=1, keepdims=True)
    ex = jnp.exp(logits - m)
    p = ex / jnp.sum(ex, axis=1, keepdims=True)

    iota_e = _iota((_T, _E), 1)
    m1 = jnp.max(p, axis=1, keepdims=True)
    i1 = jnp.min(jnp.where(p == m1, iota_e, float(_E)), axis=1, keepdims=True)
    pm = jnp.where(iota_e == i1, -1.0, p)
    m2 = jnp.max(pm, axis=1, keepdims=True)
    i2 = jnp.min(jnp.where(pm == m2, iota_e, float(_E)), axis=1, keepdims=True)

    mask0 = (iota_e == i1).astype(_f32)
    mask1 = (iota_e == i2).astype(_f32)
    masks01 = jnp.concatenate([mask0, mask1], axis=1)

    counts01 = jnp.sum(masks01, axis=0, keepdims=True)
    n0 = counts01[:, :_E]
    n = n0 + counts01[:, _E:]
    padded = jnp.ceil(n / _B) * _B

    base = jnp.zeros((1, _E), _f32)
    for k in range(1, _E):
        base = base + jnp.concatenate(
            [jnp.zeros((1, k), _f32), padded[:, :_E - k]], axis=1)

    ltri = (_iota((_T, _T), 0) > _iota((_T, _T), 1)).astype(_bf16)
    ranks01 = jnp.dot(ltri, masks01.astype(_bf16),
                      preferred_element_type=_f32)
    rank0 = jnp.sum(ranks01[:, :_E] * mask0, axis=1, keepdims=True)
    rank1 = jnp.sum((ranks01[:, _E:] + n0) * mask1, axis=1, keepdims=True)
    base0 = jnp.sum(base * mask0, axis=1, keepdims=True)
    base1 = jnp.sum(base * mask1, axis=1, keepdims=True)
    dest0 = base0 + rank0
    dest1 = base1 + rank1

    for c in range(4):
        w = _P // 4
        jrow = _iota((1, w), 1) + float(c * w)
        oht_ref[:, c * w:(c + 1) * w] = (
            (dest0 == jrow).astype(_bf16) + (dest1 == jrow).astype(_bf16))

    eq_tt = (_iota((_T, _T), 0) == _iota((_T, _T), 1)).astype(_f32)

    def flip(v):
        return jnp.sum(eq_tt * v, axis=0, keepdims=True)

    d0r, d1r, w0r, w1r = flip(dest0), flip(dest1), flip(m1), flip(m2)

    chunk = _P // 8
    for c in range(8):
        jcol = _iota((chunk, 1), 0) + float(c * chunk)
        c0 = d0r == jcol
        c1 = d1r == jcol
        oh_ref[c * chunk:(c + 1) * chunk, :] = (
            c0.astype(_bf16) + c1.astype(_bf16))
        gate_ref[c * chunk:(c + 1) * chunk, :] = jnp.sum(
            jnp.where(c0, w0r, 0.0) + jnp.where(c1, w1r, 0.0),
            axis=1, keepdims=True)

    eq_ee = (_iota((_E, _E), 0) == _iota((_E, _E), 1)).astype(_f32)
    basec = jnp.sum(eq_ee * base, axis=1, keepdims=True)
    paddedc = jnp.sum(eq_ee * padded, axis=1, keepdims=True)
    j128 = _iota((_E, 128), 1) * _B
    ind = jnp.logical_and(j128 >= basec, j128 < basec + paddedc)
    be = jnp.sum(jnp.where(ind, _iota((_E, 128), 0), 0.0),
                 axis=0, keepdims=True)
    na = jnp.sum(padded, axis=1, keepdims=True) / _B
    lastexp = jnp.max(jnp.where(padded > 0, _iota((1, _E), 1), 0.0),
                      axis=1, keepdims=True)
    jb = _iota((1, 128), 1)
    bev = jnp.where(jb < na, be, lastexp)
    meta_ref[...] = jnp.where(jb == 64.0, na, bev).astype(jnp.int32)


def _mlp_body(meta_ref, xbf_ref, oh_ref, oht_ref, gate_ref,
              w1_ref, w2_ref, out_ref, w1bf_ref, w2bf_ref):
    b = pl.program_id(0)
    na = meta_ref[64]

    @pl.when(b == 0)
    def _():
        out_ref[...] = jnp.zeros((_T, _D), _f32)

    prev = meta_ref[jnp.maximum(b, 1) - 1]
    cur = meta_ref[b]
    new_expert = jnp.logical_or(b == 0, prev != cur)

    @pl.when(jnp.logical_and(b < na, new_expert))
    def _():
        w1bf_ref[...] = w1_ref[0].astype(_bf16)
        w2bf_ref[...] = w2_ref[0].astype(_bf16)

    @pl.when(b < na)
    def _():
        xb = jnp.dot(oh_ref[...], xbf_ref[...],
                     preferred_element_type=_f32)
        h = jnp.dot(xb.astype(_bf16), w1bf_ref[...],
                    preferred_element_type=_f32)
        h = h * jax.nn.sigmoid(h)
        y = jnp.dot(h.astype(_bf16), w2bf_ref[...],
                    preferred_element_type=_f32)
        yg = (y * gate_ref[...]).astype(_bf16)
        out_ref[...] += jnp.dot(oht_ref[...], yg,
                                preferred_element_type=_f32)


@jax.jit
def kernel(x, W_router, W1, W2):
    xbf = x.astype(_bf16)
    oh, oht, gate, meta = pl.pallas_call(
        _router_body,
        out_shape=(
            jax.ShapeDtypeStruct((_P, _T), _bf16),
            jax.ShapeDtypeStruct((_T, _P), _bf16),
            jax.ShapeDtypeStruct((_P, 1), _f32),
            jax.ShapeDtypeStruct((1, 128), jnp.int32),
        ),
    )(xbf, W_router)

    grid_spec = pltpu.PrefetchScalarGridSpec(
        num_scalar_prefetch=1,
        grid=(_NB,),
        in_specs=[
            pl.BlockSpec((_T, _D), lambda b, m: (0, 0)),
            pl.BlockSpec((_B, _T), lambda b, m: (b, 0)),
            pl.BlockSpec((_T, _B), lambda b, m: (0, b)),
            pl.BlockSpec((_B, 1), lambda b, m: (b, 0)),
            pl.BlockSpec((1, _D, _H), lambda b, m: (m[b], 0, 0)),
            pl.BlockSpec((1, _H, _D), lambda b, m: (m[b], 0, 0)),
        ],
        out_specs=pl.BlockSpec((_T, _D), lambda b, m: (0, 0)),
        scratch_shapes=[
            pltpu.VMEM((_D, _H), _bf16),
            pltpu.VMEM((_H, _D), _bf16),
        ],
    )
    out = pl.pallas_call(
        _mlp_body,
        grid_spec=grid_spec,
        out_shape=jax.ShapeDtypeStruct((_T, _D), _f32),
        compiler_params=pltpu.CompilerParams(
            dimension_semantics=("arbitrary",)),
    )(meta.reshape(128), xbf, oh, oht, gate, W1, W2)
    return out


# speedup vs baseline: 1.0097x; 1.0097x over previous
import jax
import jax.numpy as jnp
from jax.experimental import pallas as pl
from jax.experimental.pallas import tpu as pltpu

_D = 1024
_E = 8
_H = 2048
_T = 1024
_B = 128
_NB = 23
_P = _NB * _B

_f32 = jnp.float32
_bf16 = jnp.bfloat16


def _iota(shape, dim):
    return jax.lax.broadcasted_iota(jnp.int32, shape, dim).astype(_f32)


def _router_body(xbf_ref, wr_ref, rid_ref, gate_ref, d0_ref, d1_ref,
                 meta_ref):
    logits = jnp.dot(xbf_ref[...], wr_ref[...].astype(_bf16),
                     preferred_element_type=_f32)
    m = jnp.max(logits, axis=1, keepdims=True)
    ex = jnp.exp(logits - m)
    p = ex / jnp.sum(ex, axis=1, keepdims=True)

    iota_e = _iota((_T, _E), 1)
    m1 = jnp.max(p, axis=1, keepdims=True)
    i1 = jnp.min(jnp.where(p == m1, iota_e, float(_E)), axis=1, keepdims=True)
    pm = jnp.where(iota_e == i1, -1.0, p)
    m2 = jnp.max(pm, axis=1, keepdims=True)
    i2 = jnp.min(jnp.where(pm == m2, iota_e, float(_E)), axis=1, keepdims=True)

    mask0 = (iota_e == i1).astype(_f32)
    mask1 = (iota_e == i2).astype(_f32)
    masks01 = jnp.concatenate([mask0, mask1], axis=1)

    counts01 = jnp.sum(masks01, axis=0, keepdims=True)
    n0 = counts01[:, :_E]
    n = n0 + counts01[:, _E:]
    padded = jnp.ceil(n / _B) * _B

    base = jnp.zeros((1, _E), _f32)
    for k in range(1, _E):
        base = base + jnp.concatenate(
            [jnp.zeros((1, k), _f32), padded[:, :_E - k]], axis=1)

    ltri = (_iota((_T, _T), 0) > _iota((_T, _T), 1)).astype(_bf16)
    ranks01 = jnp.dot(ltri, masks01.astype(_bf16),
                      preferred_element_type=_f32)
    rank0 = jnp.sum(ranks01[:, :_E] * mask0, axis=1, keepdims=True)
    rank1 = jnp.sum((ranks01[:, _E:] + n0) * mask1, axis=1, keepdims=True)
    base0 = jnp.sum(base * mask0, axis=1, keepdims=True)
    base1 = jnp.sum(base * mask1, axis=1, keepdims=True)
    dest0 = base0 + rank0
    dest1 = base1 + rank1
    d0_ref[...] = dest0
    d1_ref[...] = dest1

    eq_tt = (_iota((_T, _T), 0) == _iota((_T, _T), 1)).astype(_f32)

    def flip(v):
        return jnp.sum(eq_tt * v, axis=0, keepdims=True)

    d0r, d1r, w0r, w1r = flip(dest0), flip(dest1), flip(m1), flip(m2)

    trow = _iota((1, _T), 1)
    chunk = _P // 8
    for c in range(8):
        jcol = _iota((chunk, 1), 0) + float(c * chunk)
        c0 = d0r == jcol
        c1 = d1r == jcol
        rid_ref[c * chunk:(c + 1) * chunk, :] = jnp.sum(
            jnp.where(c0, trow, 0.0) + jnp.where(c1, trow, 0.0),
            axis=1, keepdims=True)
        gate_ref[c * chunk:(c + 1) * chunk, :] = jnp.sum(
            jnp.where(c0, w0r, 0.0) + jnp.where(c1, w1r, 0.0),
            axis=1, keepdims=True)

    eq_ee = (_iota((_E, _E), 0) == _iota((_E, _E), 1)).astype(_f32)
    basec = jnp.sum(eq_ee * base, axis=1, keepdims=True)
    paddedc = jnp.sum(eq_ee * padded, axis=1, keepdims=True)
    j128 = _iota((_E, 128), 1) * _B
    ind = jnp.logical_and(j128 >= basec, j128 < basec + paddedc)
    be = jnp.sum(jnp.where(ind, _iota((_E, 128), 0), 0.0),
                 axis=0, keepdims=True)
    na = jnp.sum(padded, axis=1, keepdims=True) / _B
    lastexp = jnp.max(jnp.where(padded > 0, _iota((1, _E), 1), 0.0),
                      axis=1, keepdims=True)
    jb = _iota((1, 128), 1)
    bev = jnp.where(jb < na, be, lastexp)
    meta_ref[...] = jnp.where(jb == 64.0, na, bev).astype(jnp.int32)


def _mlp_body(meta_ref, xbf_ref, rid_ref, gate_ref, w1_ref, w2_ref,
              yg_ref, w1bf_ref, w2bf_ref):
    b = pl.program_id(0)
    na = meta_ref[64]

    prev = meta_ref[jnp.maximum(b, 1) - 1]
    cur = meta_ref[b]
    new_expert = jnp.logical_or(b == 0, prev != cur)

    @pl.when(jnp.logical_and(b < na, new_expert))
    def _():
        w1bf_ref[...] = w1_ref[0].astype(_bf16)
        w2bf_ref[...] = w2_ref[0].astype(_bf16)

    @pl.when(b < na)
    def _():
        rid = rid_ref[...]
        onehot = (rid == _iota((_B, _T), 1)).astype(_bf16)
        xb = jnp.dot(onehot, xbf_ref[...],
                     preferred_element_type=_f32)
        h = jnp.dot(xb.astype(_bf16), w1bf_ref[...],
                    preferred_element_type=_f32)
        h = h * jax.nn.sigmoid(h)
        y = jnp.dot(h.astype(_bf16), w2bf_ref[...],
                    preferred_element_type=_f32)
        yg_ref[...] = (y * gate_ref[...]).astype(_bf16)

    @pl.when(b >= na)
    def _():
        yg_ref[...] = jnp.zeros((_B, _D), _bf16)


def _combine_body(d0_ref, d1_ref, yg_ref, out_ref):
    d0 = d0_ref[...]
    d1 = d1_ref[...]
    acc = jnp.zeros((_T, _D), _f32)
    chunk = _P // 4
    for c in range(4):
        jrow = _iota((1, chunk), 1) + float(c * chunk)
        mask = ((d0 == jrow).astype(_bf16)
                + (d1 == jrow).astype(_bf16))
        acc = acc + jnp.dot(mask, yg_ref[c * chunk:(c + 1) * chunk, :],
                            preferred_element_type=_f32)
    out_ref[...] = acc


@jax.jit
def kernel(x, W_router, W1, W2):
    xbf = x.astype(_bf16)
    rid, gate, d0, d1, meta = pl.pallas_call(
        _router_body,
        out_shape=(
            jax.ShapeDtypeStruct((_P, 1), _f32),
            jax.ShapeDtypeStruct((_P, 1), _f32),
            jax.ShapeDtypeStruct((_T, 1), _f32),
            jax.ShapeDtypeStruct((_T, 1), _f32),
            jax.ShapeDtypeStruct((1, 128), jnp.int32),
        ),
    )(xbf, W_router)

    grid_spec = pltpu.PrefetchScalarGridSpec(
        num_scalar_prefetch=1,
        grid=(_NB,),
        in_specs=[
            pl.BlockSpec((_T, _D), lambda b, m: (0, 0)),
            pl.BlockSpec((_B, 1), lambda b, m: (b, 0)),
            pl.BlockSpec((_B, 1), lambda b, m: (b, 0)),
            pl.BlockSpec((1, _D, _H), lambda b, m: (m[b], 0, 0)),
            pl.BlockSpec((1, _H, _D), lambda b, m: (m[b], 0, 0)),
        ],
        out_specs=pl.BlockSpec((_B, _D), lambda b, m: (b, 0)),
        scratch_shapes=[
            pltpu.VMEM((_D, _H), _bf16),
            pltpu.VMEM((_H, _D), _bf16),
        ],
    )
    yg = pl.pallas_call(
        _mlp_body,
        grid_spec=grid_spec,
        out_shape=jax.ShapeDtypeStruct((_P, _D), _bf16),
        compiler_params=pltpu.CompilerParams(
            dimension_semantics=("arbitrary",)),
    )(meta.reshape(128), xbf, rid, gate, W1, W2)

    out = pl.pallas_call(
        _combine_body,
        out_shape=jax.ShapeDtypeStruct((_T, _D), _f32),
    )(d0, d1, yg)
    return out


# speedup vs baseline: 1.0902x; 1.0797x over previous
import jax
import jax.numpy as jnp
from jax.experimental import pallas as pl
from jax.experimental.pallas import tpu as pltpu

_D = 1024
_E = 8
_H = 2048
_T = 1024
_B = 128
_NB = 23
_P = _NB * _B

_f32 = jnp.float32
_bf16 = jnp.bfloat16


def _iota(shape, dim):
    return jax.lax.broadcasted_iota(jnp.int32, shape, dim).astype(_f32)


def _router_body(xbf_ref, wr_ref, rid_ref, dw_ref, meta_ref):
    logits = jnp.dot(xbf_ref[...], wr_ref[...].astype(_bf16),
                     preferred_element_type=_f32)
    m = jnp.max(logits, axis=1, keepdims=True)
    ex = jnp.exp(logits - m)
    p = ex / jnp.sum(ex, axis=1, keepdims=True)

    iota_e = _iota((_T, _E), 1)
    m1 = jnp.max(p, axis=1, keepdims=True)
    i1 = jnp.min(jnp.where(p == m1, iota_e, float(_E)), axis=1, keepdims=True)
    pm = jnp.where(iota_e == i1, -1.0, p)
    m2 = jnp.max(pm, axis=1, keepdims=True)
    i2 = jnp.min(jnp.where(pm == m2, iota_e, float(_E)), axis=1, keepdims=True)

    mask0 = (iota_e == i1).astype(_f32)
    mask1 = (iota_e == i2).astype(_f32)
    masks01 = jnp.concatenate([mask0, mask1], axis=1)

    counts01 = jnp.sum(masks01, axis=0, keepdims=True)
    n0 = counts01[:, :_E]
    n = n0 + counts01[:, _E:]
    padded = jnp.ceil(n / _B) * _B

    base = jnp.zeros((1, _E), _f32)
    for k in range(1, _E):
        base = base + jnp.concatenate(
            [jnp.zeros((1, k), _f32), padded[:, :_E - k]], axis=1)

    ltri = (_iota((_T, _T), 0) > _iota((_T, _T), 1)).astype(_bf16)
    ranks01 = jnp.dot(ltri, masks01.astype(_bf16),
                      preferred_element_type=_f32)
    rank0 = jnp.sum(ranks01[:, :_E] * mask0, axis=1, keepdims=True)
    rank1 = jnp.sum((ranks01[:, _E:] + n0) * mask1, axis=1, keepdims=True)
    base0 = jnp.sum(base * mask0, axis=1, keepdims=True)
    base1 = jnp.sum(base * mask1, axis=1, keepdims=True)
    dest0 = base0 + rank0
    dest1 = base1 + rank1
    dw_ref[...] = jnp.concatenate([dest0, dest1, m1, m2], axis=1)

    eq_tt = (_iota((_T, _T), 0) == _iota((_T, _T), 1)).astype(_f32)
    d0r = jnp.sum(eq_tt * dest0, axis=0, keepdims=True)
    d1r = jnp.sum(eq_tt * dest1, axis=0, keepdims=True)

    trow = _iota((1, _T), 1)
    chunk = _P // 8
    for c in range(8):
        jcol = _iota((chunk, 1), 0) + float(c * chunk)
        rid_ref[c * chunk:(c + 1) * chunk, :] = jnp.sum(
            jnp.where(d0r == jcol, trow, 0.0)
            + jnp.where(d1r == jcol, trow, 0.0),
            axis=1, keepdims=True)

    eq_ee = (_iota((_E, _E), 0) == _iota((_E, _E), 1)).astype(_f32)
    basec = jnp.sum(eq_ee * base, axis=1, keepdims=True)
    paddedc = jnp.sum(eq_ee * padded, axis=1, keepdims=True)
    j128 = _iota((_E, 128), 1) * _B
    ind = jnp.logical_and(j128 >= basec, j128 < basec + paddedc)
    be = jnp.sum(jnp.where(ind, _iota((_E, 128), 0), 0.0),
                 axis=0, keepdims=True)
    na = jnp.sum(padded, axis=1, keepdims=True) / _B
    lastexp = jnp.max(jnp.where(padded > 0, _iota((1, _E), 1), 0.0),
                      axis=1, keepdims=True)
    jb = _iota((1, 128), 1)
    bev = jnp.where(jb < na, be, lastexp)
    meta_ref[...] = jnp.where(jb == 64.0, na, bev).astype(jnp.int32)


def _mlp_body(meta_ref, xbf_ref, rid_ref, dw_ref,
              w1e_ref, w2e_ref, out_ref, yg_ref, w1bf_ref, w2bf_ref):
    b = pl.program_id(0)
    na = meta_ref[64]

    prev = meta_ref[jnp.maximum(b, 1) - 1]
    cur = meta_ref[jnp.minimum(b, _NB - 1)]
    new_expert = jnp.logical_or(b == 0, prev != cur)

    @pl.when(jnp.logical_and(b < na, new_expert))
    def _():
        w1bf_ref[...] = w1e_ref[0].astype(_bf16)
        w2bf_ref[...] = w2e_ref[0].astype(_bf16)

    @pl.when(b < na)
    def _():
        rid = rid_ref[...]
        onehot = (rid == _iota((_B, _T), 1)).astype(_bf16)
        xb = jnp.dot(onehot, xbf_ref[...],
                     preferred_element_type=_f32)
        h = jnp.dot(xb.astype(_bf16), w1bf_ref[...],
                    preferred_element_type=_f32)
        h = h * jax.nn.sigmoid(h)
        y = jnp.dot(h.astype(_bf16), w2bf_ref[...],
                    preferred_element_type=_f32)
        yg_ref[pl.ds(b * _B, _B), :] = y.astype(_bf16)

    @pl.when(jnp.logical_and(b >= na, b < _NB))
    def _():
        yg_ref[pl.ds(b * _B, _B), :] = jnp.zeros((_B, _D), _bf16)

    @pl.when(b == _NB)
    def _():
        dw = dw_ref[...]
        d0 = dw[:, 0:1]
        d1 = dw[:, 1:2]
        w0 = dw[:, 2:3].astype(_bf16)
        w1 = dw[:, 3:4].astype(_bf16)
        chunk = _P // 4
        for c in range(4):
            jrow = _iota((1, chunk), 1) + float(c * chunk)
            mask = (jnp.where(d0 == jrow, 1.0, 0.0).astype(_bf16) * w0
                    + jnp.where(d1 == jrow, 1.0, 0.0).astype(_bf16) * w1)
            contrib = jnp.dot(mask, yg_ref[c * chunk:(c + 1) * chunk, :],
                              preferred_element_type=_f32)
            if c == 0:
                out_ref[...] = contrib
            else:
                out_ref[...] += contrib


@jax.jit
def kernel(x, W_router, W1, W2):
    xbf = x.astype(_bf16)
    rid, dw, meta = pl.pallas_call(
        _router_body,
        out_shape=(
            jax.ShapeDtypeStruct((_P, 1), _f32),
            jax.ShapeDtypeStruct((_T, 4), _f32),
            jax.ShapeDtypeStruct((1, 128), jnp.int32),
        ),
    )(xbf, W_router)

    grid_spec = pltpu.PrefetchScalarGridSpec(
        num_scalar_prefetch=1,
        grid=(_NB + 1,),
        in_specs=[
            pl.BlockSpec((_T, _D), lambda b, m: (0, 0)),
            pl.BlockSpec((_B, 1),
                         lambda b, m: (jnp.minimum(b, _NB - 1), 0)),
            pl.BlockSpec((_T, 4), lambda b, m: (0, 0)),
            pl.BlockSpec((1, _D, _H), lambda b, m: (m[b], 0, 0)),
            pl.BlockSpec((1, _H, _D), lambda b, m: (m[b], 0, 0)),
        ],
        out_specs=pl.BlockSpec((_T, _D), lambda b, m: (0, 0)),
        scratch_shapes=[
            pltpu.VMEM((_P, _D), _bf16),
            pltpu.VMEM((_D, _H), _bf16),
            pltpu.VMEM((_H, _D), _bf16),
        ],
    )
    out = pl.pallas_call(
        _mlp_body,
        grid_spec=grid_spec,
        out_shape=jax.ShapeDtypeStruct((_T, _D), _f32),
        compiler_params=pltpu.CompilerParams(
            dimension_semantics=("arbitrary",)),
    )(meta.reshape(128), xbf, rid, dw, W1, W2)
    return out
